# baseline (speedup 1.0000x reference)
import functools

import jax
import jax.numpy as jnp
from jax import lax
from jax.experimental import pallas as pl
from jax.experimental.pallas import tpu as pltpu
from jax.experimental.pallas import tpu_sc as plsc

VOCAB = 100002
EMB_DIM = 128
BATCH = 1024
SEQ = 200
PAD_IX = 0

_NC = 2
_NS = 16
_NW = _NC * _NS

_N = BATCH * SEQ
_BPW = _N // _NW
_CHUNK = 160
_NCHUNK = _BPW // _CHUNK
_NBUF = 4
_NGROUP = _NCHUNK // _NBUF


def _sc_body(table_hbm, idx_hbm, out_hbm,
             idx_v, rows0, rows1, rows2, rows3,
             gsem0, gsem1, gsem2, gsem3):
    wid = lax.axis_index("s") * _NC + lax.axis_index("c")
    base = wid * _BPW

    pltpu.sync_copy(idx_hbm.at[pl.ds(base, _BPW)], idx_v)

    rows = (rows0, rows1, rows2, rows3)
    gsem = (gsem0, gsem1, gsem2, gsem3)

    def g_desc(c, b):
        return pltpu.make_async_copy(
            table_hbm.at[idx_v.at[pl.ds(c * _CHUNK, _CHUNK)]], rows[b], gsem[b])

    def wb(c, b):
        pltpu.sync_copy(rows[b], out_hbm.at[pl.ds(base + c * _CHUNK, _CHUNK)])

    for b in range(_NBUF):
        g_desc(b, b).start()

    def group_body(p, carry):
        c0 = p * _NBUF
        for b in range(_NBUF):
            g_desc(c0 + b, b).wait()
            wb(c0 + b, b)
            g_desc(c0 + b + _NBUF, b).start()
        return carry

    lax.fori_loop(0, _NGROUP - 1, group_body, 0)

    c0 = (_NGROUP - 1) * _NBUF
    for b in range(_NBUF):
        g_desc(c0 + b, b).wait()
        wb(c0 + b, b)


def _mask_body(enc_ref, mask_ref):
    mask_ref[...] = jnp.where(enc_ref[...] != PAD_IX, 1, 0).astype(jnp.int32)


@jax.jit
def _embed(table, idx, encoded):
    mesh = plsc.VectorSubcoreMesh(core_axis_name="c", subcore_axis_name="s")
    sc = functools.partial(
        pl.kernel,
        out_type=jax.ShapeDtypeStruct((_N, EMB_DIM), jnp.float32),
        mesh=mesh,
        scratch_types=[
            pltpu.VMEM((_BPW,), jnp.int32),
            pltpu.VMEM((_CHUNK, EMB_DIM), jnp.float32),
            pltpu.VMEM((_CHUNK, EMB_DIM), jnp.float32),
            pltpu.VMEM((_CHUNK, EMB_DIM), jnp.float32),
            pltpu.VMEM((_CHUNK, EMB_DIM), jnp.float32),
            pltpu.SemaphoreType.DMA,
            pltpu.SemaphoreType.DMA,
            pltpu.SemaphoreType.DMA,
            pltpu.SemaphoreType.DMA,
        ],
    )(_sc_body)
    out_flat = sc(table, idx)
    mask = pl.pallas_call(
        _mask_body,
        out_shape=jax.ShapeDtypeStruct((BATCH, SEQ), jnp.int32),
    )(encoded)
    return out_flat, mask


def kernel(encoded, table):
    idx = encoded.reshape(_N)
    out_flat, mask = _embed(table, idx, encoded)
    return out_flat.reshape(BATCH, SEQ, EMB_DIM), mask, encoded


# speedup vs baseline: 1.0088x; 1.0088x over previous
import functools

import jax
import jax.numpy as jnp
from jax import lax
from jax.experimental import pallas as pl
from jax.experimental.pallas import tpu as pltpu
from jax.experimental.pallas import tpu_sc as plsc

VOCAB = 100002
EMB_DIM = 128
BATCH = 1024
SEQ = 200
PAD_IX = 0

_NC = 2
_NS = 16
_NW = _NC * _NS

_RPW = BATCH // _NW
_NBUF = 4
_NGROUP = _RPW // _NBUF
_S0 = 128
_S1 = SEQ - _S0


def _sc_body(table_hbm, enc_hbm, out_hbm,
             idx_v, rows0, rows1, rows2, rows3,
             gsem0, gsem1, gsem2, gsem3):
    wid = lax.axis_index("s") * _NC + lax.axis_index("c")
    rbase = wid * _RPW
    fbase = wid * _RPW * SEQ

    pltpu.sync_copy(enc_hbm.at[pl.ds(rbase, _RPW)], idx_v)

    rows = (rows0, rows1, rows2, rows3)
    gsem = (gsem0, gsem1, gsem2, gsem3)

    def g0(c, b):
        return pltpu.make_async_copy(
            table_hbm.at[idx_v.at[c, pl.ds(0, _S0)]],
            rows[b].at[pl.ds(0, _S0)], gsem[b])

    def g1(c, b):
        return pltpu.make_async_copy(
            table_hbm.at[idx_v.at[c, pl.ds(_S0, _S1)]],
            rows[b].at[pl.ds(_S0, _S1)], gsem[b])

    def wb(c, b):
        pltpu.sync_copy(rows[b], out_hbm.at[pl.ds(fbase + c * SEQ, SEQ)])

    for b in range(_NBUF):
        g0(b, b).start()
        g1(b, b).start()

    def group_body(p, carry):
        c0 = p * _NBUF
        for b in range(_NBUF):
            g0(c0 + b, b).wait()
            g1(c0 + b, b).wait()
            wb(c0 + b, b)
            g0(c0 + b + _NBUF, b).start()
            g1(c0 + b + _NBUF, b).start()
        return carry

    lax.fori_loop(0, _NGROUP - 1, group_body, 0)

    c0 = (_NGROUP - 1) * _NBUF
    for b in range(_NBUF):
        g0(c0 + b, b).wait()
        g1(c0 + b, b).wait()
        wb(c0 + b, b)


def _mask_body(enc_ref, mask_ref):
    mask_ref[...] = jnp.where(enc_ref[...] != PAD_IX, 1, 0).astype(jnp.int32)


@jax.jit
def _embed(table, encoded):
    mesh = plsc.VectorSubcoreMesh(core_axis_name="c", subcore_axis_name="s")
    sc = functools.partial(
        pl.kernel,
        out_type=jax.ShapeDtypeStruct((BATCH * SEQ, EMB_DIM), jnp.float32),
        mesh=mesh,
        scratch_types=[
            pltpu.VMEM((_RPW, SEQ), jnp.int32),
            pltpu.VMEM((SEQ, EMB_DIM), jnp.float32),
            pltpu.VMEM((SEQ, EMB_DIM), jnp.float32),
            pltpu.VMEM((SEQ, EMB_DIM), jnp.float32),
            pltpu.VMEM((SEQ, EMB_DIM), jnp.float32),
            pltpu.SemaphoreType.DMA,
            pltpu.SemaphoreType.DMA,
            pltpu.SemaphoreType.DMA,
            pltpu.SemaphoreType.DMA,
        ],
        compiler_params=pltpu.CompilerParams(use_tc_tiling_on_sc=True),
    )(_sc_body)
    out_flat = sc(table, encoded)
    mask = pl.pallas_call(
        _mask_body,
        out_shape=jax.ShapeDtypeStruct((BATCH, SEQ), jnp.int32),
    )(encoded)
    return out_flat, mask


def kernel(encoded, table):
    out_flat, mask = _embed(table, encoded)
    return out_flat.reshape(BATCH, SEQ, EMB_DIM), mask, encoded
